# baseline (speedup 1.0000x reference)
import functools

import jax
import jax.numpy as jnp
from jax import lax
from jax.experimental import pallas as pl
from jax.experimental.pallas import tpu as pltpu
from jax.experimental.pallas import tpu_sc as plsc

B, N, K = 16, 20000, 12000
L = 16
C = N // L
RADIX = 256
NBUF = N + L
IMG_W, IMG_H = 768.0, 432.0


def _tc_prep(scores3, rps_t):
    def body(s_ref, rp_ref, keys_ref, rois_ref):
        rp = rp_ref[0]
        x = rp[0:1, :]
        y = rp[1:2, :]
        w = rp[2:3, :]
        h = rp[3:4, :]
        x1 = jnp.clip(x - w * 0.5, 0.0, IMG_W)
        x2 = jnp.clip(x + w * 0.5, 0.0, IMG_W)
        y1 = jnp.clip(y - h * 0.5, 0.0, IMG_H)
        y2 = jnp.clip(y + h * 0.5, 0.0, IMG_H)
        wn = x2 - x1
        hn = y2 - y1
        rois_ref[0, 0:1, :] = lax.bitcast_convert_type(x1 + wn * 0.5, jnp.int32)
        rois_ref[0, 1:2, :] = lax.bitcast_convert_type(y1 + hn * 0.5, jnp.int32)
        rois_ref[0, 2:3, :] = lax.bitcast_convert_type(wn, jnp.int32)
        rois_ref[0, 3:4, :] = lax.bitcast_convert_type(hn, jnp.int32)
        s = s_ref[0]
        masked = jnp.where((wn > 16.0) & (hn > 16.0), s, 0.0)
        keys_ref[0] = lax.bitcast_convert_type(masked, jnp.int32)

    return pl.pallas_call(
        body,
        grid=(B,),
        in_specs=[
            pl.BlockSpec((1, 1, N), lambda b: (b, 0, 0)),
            pl.BlockSpec((1, 4, N), lambda b: (b, 0, 0)),
        ],
        out_specs=[
            pl.BlockSpec((1, 1, N), lambda b: (b, 0, 0)),
            pl.BlockSpec((1, 4, N), lambda b: (b, 0, 0)),
        ],
        out_shape=[
            jax.ShapeDtypeStruct((B, 1, N), jnp.int32),
            jax.ShapeDtypeStruct((B, 4, N), jnp.int32),
        ],
    )(scores3, rps_t)


def _sc_sort_gather(keys_flat, rois_flat):
    mesh = plsc.VectorSubcoreMesh(core_axis_name="c", subcore_axis_name="s")

    @functools.partial(
        pl.kernel,
        mesh=mesh,
        compiler_params=pltpu.CompilerParams(needs_layout_passes=False),
        out_type=[
            jax.ShapeDtypeStruct((B * K,), jnp.int32),
            jax.ShapeDtypeStruct((4 * B * K,), jnp.int32),
        ],
        scratch_types=[
            pltpu.VMEM((NBUF,), jnp.int32),
            pltpu.VMEM((NBUF,), jnp.int32),
            pltpu.VMEM((NBUF,), jnp.int32),
            pltpu.VMEM((NBUF,), jnp.int32),
            pltpu.VMEM((RADIX * L,), jnp.int32),
        ],
    )
    def k(keys_hbm, rois_hbm, skey_out, rois_out,
          key_a, key_b, idx_a, idx_b, hist):
        cid = lax.axis_index("c")
        sid = lax.axis_index("s")
        wid = sid * 2 + cid
        lane = lax.iota(jnp.int32, 16)
        ones = jnp.ones((16,), jnp.int32)
        zeros = jnp.zeros((16,), jnp.int32)

        @pl.when(wid < B)
        def _():
            b = wid
            pltpu.sync_copy(keys_hbm.at[pl.ds(b * N, N)],
                            key_a.at[pl.ds(0, N)])

            def init_body(j, _):
                idx_a[pl.ds(j * 16, 16)] = lane + j * 16
                return 0

            lax.fori_loop(0, C, init_body, 0)

            def zero_hist():
                def zero_body(t, _):
                    hist[pl.ds(t * 16, 16)] = zeros
                    return 0

                lax.fori_loop(0, RADIX, zero_body, 0)

            zero_hist()

            def phist_body(j, _):
                g = lane * C + j
                kk = plsc.load_gather(key_a, [g])
                d = jnp.minimum(lax.shift_right_logical(kk, 22), RADIX - 1)
                plsc.addupdate_scatter(hist, [d * L + lane], ones)
                return 0

            lax.fori_loop(0, C, phist_body, 0)

            def dscan_body(t, carry):
                acc, dstar, keepv = carry
                d = RADIX - 1 - t
                row = hist[pl.ds(d * 16, 16)]
                cum = plsc.cumsum(row)
                acc_new = acc + cum[15]
                rowacc = keepv + row
                crossed = (acc_new >= K) & (acc < K)
                dstar = jnp.where(crossed, d, dstar)
                keepv = jnp.where(acc < K, rowacc, keepv)
                return acc_new, dstar, keepv

            _, dstar, keepv = lax.fori_loop(
                0, RADIX, dscan_body, (jnp.int32(0), jnp.int32(0), zeros))

            keep_cum = plsc.cumsum(keepv)
            m_cnt = keep_cum[15]

            def compact_body(j, cursor):
                g = lane * C + j
                kk = plsc.load_gather(key_a, [g])
                iv = plsc.load_gather(idx_a, [g])
                mk = jnp.minimum(lax.shift_right_logical(kk, 22),
                                 RADIX - 1) >= dstar
                plsc.store_scatter(key_b, [cursor], kk, mask=mk)
                plsc.store_scatter(idx_b, [cursor], iv, mask=mk)
                return cursor + mk.astype(jnp.int32)

            lax.fori_loop(0, C, compact_body, keep_cum - keepv)

            plsc.store_scatter(key_b, [m_cnt + lane], zeros)
            plsc.store_scatter(idx_b, [m_cnt + lane], zeros)
            cm = lax.shift_right_logical(m_cnt + 15, 4)

            def radix_pass(src_k, src_i, dst_k, dst_i, shift):
                zero_hist()

                def hist_body(j, _):
                    g = lane * cm + j
                    kk = plsc.load_gather(src_k, [g])
                    d = lax.shift_right_logical(~kk, shift) & (RADIX - 1)
                    plsc.addupdate_scatter(hist, [d * L + lane], ones)
                    return 0

                lax.fori_loop(0, cm, hist_body, 0)

                def off_body(t, carry):
                    row = hist[pl.ds(t * 16, 16)]
                    cum = plsc.cumsum(row)
                    hist[pl.ds(t * 16, 16)] = (cum - row) + carry
                    return carry + cum[15]

                lax.fori_loop(0, RADIX, off_body, jnp.int32(0))

                def perm_body(j, _):
                    g = lane * cm + j
                    kk = plsc.load_gather(src_k, [g])
                    iv = plsc.load_gather(src_i, [g])
                    d = lax.shift_right_logical(~kk, shift) & (RADIX - 1)
                    slot = d * L + lane
                    pos = plsc.load_gather(hist, [slot])
                    plsc.store_scatter(dst_k, [pos], kk)
                    plsc.store_scatter(dst_i, [pos], iv)
                    plsc.addupdate_scatter(hist, [slot], ones)
                    return 0

                lax.fori_loop(0, cm, perm_body, 0)

            radix_pass(key_b, idx_b, key_a, idx_a, 0)
            radix_pass(key_a, idx_a, key_b, idx_b, 8)
            radix_pass(key_b, idx_b, key_a, idx_a, 16)
            radix_pass(key_a, idx_a, key_b, idx_b, 24)

            pltpu.sync_copy(key_b.at[pl.ds(0, K)],
                            skey_out.at[pl.ds(b * K, K)])

            for c in range(4):
                pltpu.sync_copy(rois_hbm.at[pl.ds((b * 4 + c) * N, N)],
                                key_a.at[pl.ds(0, N)])

                def gather_body(j, _):
                    iv = idx_b[pl.ds(j * 16, 16)]
                    idx_a[pl.ds(j * 16, 16)] = plsc.load_gather(key_a, [iv])
                    return 0

                lax.fori_loop(0, K // 16, gather_body, 0)
                pltpu.sync_copy(idx_a.at[pl.ds(0, K)],
                                rois_out.at[pl.ds((c * B + b) * K, K)])

    return k(keys_flat, rois_flat)


def kernel(scores, rps, n_train_pre_nms):
    del n_train_pre_nms
    scores3 = scores.reshape(B, 1, N)
    rps_t = jnp.swapaxes(rps, 1, 2)
    keys3, rois_planes = _tc_prep(scores3, rps_t)
    skey, rois_bits = _sc_sort_gather(
        keys3.reshape(B * N), rois_planes.reshape(B * 4 * N))
    scores_out = lax.bitcast_convert_type(skey, jnp.float32).reshape(B, K, 1)
    rois_out = jnp.transpose(
        lax.bitcast_convert_type(rois_bits, jnp.float32).reshape(4, B, K),
        (1, 2, 0))
    return rois_out, scores_out


# speedup vs baseline: 1.2461x; 1.2461x over previous
import functools

import jax
import jax.numpy as jnp
from jax import lax
from jax.experimental import pallas as pl
from jax.experimental.pallas import tpu as pltpu
from jax.experimental.pallas import tpu_sc as plsc

B, N, K = 16, 20000, 12000
L = 16
C = N // L
RADIX = 256
IMG_W, IMG_H = 768.0, 432.0


def _tc_prep(scores3, rps_t):
    def body(s_ref, rp_ref, keys_ref, rois_ref):
        rp = rp_ref[0]
        x = rp[0:1, :]
        y = rp[1:2, :]
        w = rp[2:3, :]
        h = rp[3:4, :]
        x1 = jnp.clip(x - w * 0.5, 0.0, IMG_W)
        x2 = jnp.clip(x + w * 0.5, 0.0, IMG_W)
        y1 = jnp.clip(y - h * 0.5, 0.0, IMG_H)
        y2 = jnp.clip(y + h * 0.5, 0.0, IMG_H)
        wn = x2 - x1
        hn = y2 - y1
        rois_ref[0, 0:1, :] = lax.bitcast_convert_type(x1 + wn * 0.5, jnp.int32)
        rois_ref[0, 1:2, :] = lax.bitcast_convert_type(y1 + hn * 0.5, jnp.int32)
        rois_ref[0, 2:3, :] = lax.bitcast_convert_type(wn, jnp.int32)
        rois_ref[0, 3:4, :] = lax.bitcast_convert_type(hn, jnp.int32)
        s = s_ref[0]
        masked = jnp.where((wn > 16.0) & (hn > 16.0), s, 0.0)
        keys_ref[0] = lax.bitcast_convert_type(masked, jnp.int32)

    return pl.pallas_call(
        body,
        grid=(B,),
        in_specs=[
            pl.BlockSpec((1, 1, N), lambda b: (b, 0, 0)),
            pl.BlockSpec((1, 4, N), lambda b: (b, 0, 0)),
        ],
        out_specs=[
            pl.BlockSpec((1, 1, N), lambda b: (b, 0, 0)),
            pl.BlockSpec((1, 4, N), lambda b: (b, 0, 0)),
        ],
        out_shape=[
            jax.ShapeDtypeStruct((B, 1, N), jnp.int32),
            jax.ShapeDtypeStruct((B, 4, N), jnp.int32),
        ],
    )(scores3, rps_t)


def _sc_sort_gather(keys_flat, rois_flat):
    mesh = plsc.VectorSubcoreMesh(core_axis_name="c", subcore_axis_name="s")

    @functools.partial(
        pl.kernel,
        mesh=mesh,
        compiler_params=pltpu.CompilerParams(needs_layout_passes=False),
        out_type=[
            jax.ShapeDtypeStruct((B * K,), jnp.int32),
            jax.ShapeDtypeStruct((4 * B * K,), jnp.int32),
        ],
        scratch_types=[
            pltpu.VMEM((N,), jnp.int32),
            pltpu.VMEM((N,), jnp.int32),
            pltpu.VMEM((N,), jnp.int32),
            pltpu.VMEM((N,), jnp.int32),
            pltpu.VMEM((RADIX * L,), jnp.int32),
        ],
    )
    def k(keys_hbm, rois_hbm, skey_out, rois_out,
          key_a, key_b, idx_a, idx_b, hist):
        cid = lax.axis_index("c")
        sid = lax.axis_index("s")
        wid = sid * 2 + cid
        lane = lax.iota(jnp.int32, 16)
        ones = jnp.ones((16,), jnp.int32)

        @pl.when(wid < B)
        def _():
            b = wid
            pltpu.sync_copy(keys_hbm.at[pl.ds(b * N, N)], key_a)

            def init_body(j, _):
                idx_a[pl.ds(j * 16, 16)] = lane + j * 16
                return 0

            lax.fori_loop(0, C, init_body, 0)

            def radix_pass(src_k, src_i, dst_k, dst_i, shift):
                def zero_body(t, _):
                    hist[pl.ds(t * 16, 16)] = jnp.zeros((16,), jnp.int32)
                    return 0

                lax.fori_loop(0, RADIX, zero_body, 0)

                def hist_body(j, _):
                    g = lane * C + j
                    kk = plsc.load_gather(src_k, [g])
                    d = lax.shift_right_logical(~kk, shift) & (RADIX - 1)
                    plsc.addupdate_scatter(hist, [d * L + lane], ones)
                    return 0

                lax.fori_loop(0, C, hist_body, 0)

                def off_body(t, carry):
                    row = hist[pl.ds(t * 16, 16)]
                    cum = plsc.cumsum(row)
                    hist[pl.ds(t * 16, 16)] = (cum - row) + carry
                    return carry + cum[15]

                lax.fori_loop(0, RADIX, off_body, jnp.int32(0))

                def perm_body(j, _):
                    g = lane * C + j
                    kk = plsc.load_gather(src_k, [g])
                    iv = plsc.load_gather(src_i, [g])
                    d = lax.shift_right_logical(~kk, shift) & (RADIX - 1)
                    slot = d * L + lane
                    pos = plsc.load_gather(hist, [slot])
                    plsc.store_scatter(dst_k, [pos], kk)
                    plsc.store_scatter(dst_i, [pos], iv)
                    plsc.addupdate_scatter(hist, [slot], ones)
                    return 0

                lax.fori_loop(0, C, perm_body, 0)

            radix_pass(key_a, idx_a, key_b, idx_b, 0)
            radix_pass(key_b, idx_b, key_a, idx_a, 8)
            radix_pass(key_a, idx_a, key_b, idx_b, 16)
            radix_pass(key_b, idx_b, key_a, idx_a, 24)

            pltpu.sync_copy(key_a.at[pl.ds(0, K)],
                            skey_out.at[pl.ds(b * K, K)])

            for c in range(4):
                pltpu.sync_copy(rois_hbm.at[pl.ds((b * 4 + c) * N, N)],
                                key_b)

                def gather_body(j, _):
                    iv = idx_a[pl.ds(j * 16, 16)]
                    idx_b[pl.ds(j * 16, 16)] = plsc.load_gather(key_b, [iv])
                    return 0

                lax.fori_loop(0, K // 16, gather_body, 0)
                pltpu.sync_copy(idx_b.at[pl.ds(0, K)],
                                rois_out.at[pl.ds((c * B + b) * K, K)])

    return k(keys_flat, rois_flat)


def kernel(scores, rps, n_train_pre_nms):
    del n_train_pre_nms
    scores3 = scores.reshape(B, 1, N)
    rps_t = jnp.swapaxes(rps, 1, 2)
    keys3, rois_planes = _tc_prep(scores3, rps_t)
    skey, rois_bits = _sc_sort_gather(
        keys3.reshape(B * N), rois_planes.reshape(B * 4 * N))
    scores_out = lax.bitcast_convert_type(skey, jnp.float32).reshape(B, K, 1)
    rois_out = jnp.transpose(
        lax.bitcast_convert_type(rois_bits, jnp.float32).reshape(4, B, K),
        (1, 2, 0))
    return rois_out, scores_out
